# baseline (speedup 1.0000x reference)
import jax
import jax.numpy as jnp
from jax.experimental import pallas as pl
from jax.experimental.pallas import tpu as pltpu

TOKENS = 2048
HIDDEN = 2048
INTER = 2048
NUM_EXPERTS = 8
TOPK = 2

TM = 256
TN1 = 512
TN2 = 1024
NPAIRS = TOKENS * TOPK
NPAD = NPAIRS + NUM_EXPERTS * TM
NTILES = NPAD // TM
NP1 = INTER // TN1
NP2 = HIDDEN // TN2
NPH = NP1 + NP2

_PREC = jax.lax.Precision.DEFAULT


def _moe_body(te_ref, act_ref, x_ref, wg_ref, wu_ref, w2_ref, o_ref, h_ref):
    t = pl.program_id(0)
    p = pl.program_id(1)

    @pl.when(act_ref[t] != 0)
    def _():
        @pl.when(p < NP1)
        def _():
            x = x_ref[...].astype(jnp.bfloat16)
            g = jax.lax.dot_general(x, wg_ref[0].astype(jnp.bfloat16),
                                    (((1,), (1,)), ((), ())),
                                    preferred_element_type=jnp.float32,
                                    precision=_PREC)
            u = jax.lax.dot_general(x, wu_ref[0].astype(jnp.bfloat16),
                                    (((1,), (1,)), ((), ())),
                                    preferred_element_type=jnp.float32,
                                    precision=_PREC)
            h_ref[:, pl.ds(p * TN1, TN1)] = (
                (g * jax.nn.sigmoid(g)) * u).astype(jnp.bfloat16)

        @pl.when(p >= NP1)
        def _():
            h = h_ref[...]
            o_ref[...] = jax.lax.dot_general(
                h, w2_ref[0].astype(jnp.bfloat16), (((1,), (1,)), ((), ())),
                preferred_element_type=jnp.float32, precision=_PREC)


@jax.jit
def kernel(hidden_states, router_logits, w13_weight, w2_weight):
    orig_shape = hidden_states.shape
    hs = hidden_states.reshape(TOKENS, HIDDEN)
    gating = router_logits.reshape(TOKENS, NUM_EXPERTS)

    probs = jax.nn.softmax(gating.astype(jnp.float32), axis=-1)
    topk_w, topk_i = jax.lax.top_k(probs, TOPK)
    topk_w = topk_w / topk_w.sum(axis=-1, keepdims=True)

    flat_e = topk_i.reshape(-1).astype(jnp.int32)

    onehot = (flat_e[:, None] == jnp.arange(NUM_EXPERTS, dtype=jnp.int32)[None, :])
    counts = onehot.sum(axis=0, dtype=jnp.int32)
    rank = jnp.take_along_axis(jnp.cumsum(onehot, axis=0, dtype=jnp.int32) - 1,
                               flat_e[:, None], axis=1)[:, 0]
    padded_counts = ((counts + TM - 1) // TM) * TM
    cend = jnp.cumsum(padded_counts)
    padded_off = cend - padded_counts
    pos = padded_off[flat_e] + rank

    tile_start = jnp.arange(NTILES, dtype=jnp.int32) * TM
    tile_expert = jnp.minimum(
        (tile_start[:, None] >= cend[None, :]).sum(axis=1, dtype=jnp.int32),
        NUM_EXPERTS - 1)
    tile_active = (tile_start - padded_off[tile_expert]
                   < counts[tile_expert]).astype(jnp.int32)

    token_id = jnp.arange(NPAIRS, dtype=jnp.int32) // TOPK
    src_row = jnp.zeros((NPAD,), jnp.int32).at[pos].set(token_id)
    x_pad = hs[src_row]

    y_pad = pl.pallas_call(
        _moe_body,
        grid_spec=pltpu.PrefetchScalarGridSpec(
            num_scalar_prefetch=2,
            grid=(NTILES, NPH),
            in_specs=[
                pl.BlockSpec((TM, HIDDEN), lambda t, p, te, act: (t, 0)),
                pl.BlockSpec((1, TN1, HIDDEN),
                             lambda t, p, te, act:
                             (te[t], jnp.minimum(p, NP1 - 1), 0)),
                pl.BlockSpec((1, TN1, HIDDEN),
                             lambda t, p, te, act:
                             (te[t], jnp.minimum(p, NP1 - 1) + NP1, 0)),
                pl.BlockSpec((1, TN2, INTER),
                             lambda t, p, te, act:
                             (te[t], jnp.maximum(p - NP1, 0), 0)),
            ],
            out_specs=pl.BlockSpec((TM, TN2),
                                   lambda t, p, te, act:
                                   (t, jnp.maximum(p - NP1, 0))),
            scratch_shapes=[pltpu.VMEM((TM, INTER), jnp.bfloat16)],
        ),
        out_shape=jax.ShapeDtypeStruct((NPAD, HIDDEN), jnp.float32),
        compiler_params=pltpu.CompilerParams(
            dimension_semantics=("arbitrary", "arbitrary")),
    )(tile_expert, tile_active, x_pad, w13_weight, w13_weight, w2_weight)

    y = y_pad[pos].reshape(TOKENS, TOPK, HIDDEN)
    out = (y * topk_w[:, :, None]).sum(axis=1)
    return out.reshape(orig_shape)


# speedup vs baseline: 1.2388x; 1.2388x over previous
import functools

import jax
import jax.numpy as jnp
from jax.experimental import pallas as pl
from jax.experimental.pallas import tpu as pltpu

TOKENS = 2048
HIDDEN = 2048
INTER = 2048
NUM_EXPERTS = 8
TOPK = 2

TM = 256
TN = 1024
NPAIRS = TOKENS * TOPK
NPAD = NPAIRS + NUM_EXPERTS * TM
NTILES = NPAD // TM

_PREC = jax.lax.Precision.DEFAULT


def _gmm1_body(te_ref, act_ref, x_ref, wg_ref, wu_ref, o_ref):
    t = pl.program_id(1)

    @pl.when(act_ref[t] != 0)
    def _():
        x = x_ref[...].astype(jnp.bfloat16)
        g = jax.lax.dot_general(x, wg_ref[0].astype(jnp.bfloat16),
                                (((1,), (1,)), ((), ())),
                                preferred_element_type=jnp.float32,
                                precision=_PREC)
        u = jax.lax.dot_general(x, wu_ref[0].astype(jnp.bfloat16),
                                (((1,), (1,)), ((), ())),
                                preferred_element_type=jnp.float32,
                                precision=_PREC)
        o_ref[...] = (g * jax.nn.sigmoid(g)) * u


def _gmm2_body(te_ref, act_ref, h_ref, w_ref, o_ref):
    t = pl.program_id(1)

    @pl.when(act_ref[t] != 0)
    def _():
        h = h_ref[...].astype(jnp.bfloat16)
        o_ref[...] = jax.lax.dot_general(h, w_ref[0].astype(jnp.bfloat16),
                                         (((1,), (1,)), ((), ())),
                                         preferred_element_type=jnp.float32,
                                         precision=_PREC)


@jax.jit
def kernel(hidden_states, router_logits, w13_weight, w2_weight):
    orig_shape = hidden_states.shape
    hs = hidden_states.reshape(TOKENS, HIDDEN)
    gating = router_logits.reshape(TOKENS, NUM_EXPERTS)

    probs = jax.nn.softmax(gating.astype(jnp.float32), axis=-1)
    topk_w, topk_i = jax.lax.top_k(probs, TOPK)
    topk_w = topk_w / topk_w.sum(axis=-1, keepdims=True)

    flat_e = topk_i.reshape(-1).astype(jnp.int32)

    onehot = (flat_e[:, None] == jnp.arange(NUM_EXPERTS, dtype=jnp.int32)[None, :])
    counts = onehot.sum(axis=0, dtype=jnp.int32)
    rank = jnp.take_along_axis(jnp.cumsum(onehot, axis=0, dtype=jnp.int32) - 1,
                               flat_e[:, None], axis=1)[:, 0]
    padded_counts = ((counts + TM - 1) // TM) * TM
    cend = jnp.cumsum(padded_counts)
    padded_off = cend - padded_counts
    pos = padded_off[flat_e] + rank

    tile_start = jnp.arange(NTILES, dtype=jnp.int32) * TM
    tile_expert = jnp.minimum(
        (tile_start[:, None] >= cend[None, :]).sum(axis=1, dtype=jnp.int32),
        NUM_EXPERTS - 1)
    tile_active = (tile_start - padded_off[tile_expert]
                   < counts[tile_expert]).astype(jnp.int32)

    token_id = jnp.arange(NPAIRS, dtype=jnp.int32) // TOPK
    src_row = jnp.zeros((NPAD,), jnp.int32).at[pos].set(token_id)
    x_pad = hs[src_row]

    n1 = INTER // TN
    h_pad = pl.pallas_call(
        _gmm1_body,
        grid_spec=pltpu.PrefetchScalarGridSpec(
            num_scalar_prefetch=2,
            grid=(n1, NTILES),
            in_specs=[
                pl.BlockSpec((TM, HIDDEN), lambda n, t, te, act: (t, 0)),
                pl.BlockSpec((1, TN, HIDDEN), lambda n, t, te, act: (te[t], n, 0)),
                pl.BlockSpec((1, TN, HIDDEN),
                             lambda n, t, te, act: (te[t], n + INTER // TN, 0)),
            ],
            out_specs=pl.BlockSpec((TM, TN), lambda n, t, te, act: (t, n)),
        ),
        out_shape=jax.ShapeDtypeStruct((NPAD, INTER), jnp.float32),
        compiler_params=pltpu.CompilerParams(
            dimension_semantics=("arbitrary", "arbitrary")),
    )(tile_expert, tile_active, x_pad, w13_weight, w13_weight)

    n2 = HIDDEN // TN
    y_pad = pl.pallas_call(
        _gmm2_body,
        grid_spec=pltpu.PrefetchScalarGridSpec(
            num_scalar_prefetch=2,
            grid=(n2, NTILES),
            in_specs=[
                pl.BlockSpec((TM, INTER), lambda n, t, te, act: (t, 0)),
                pl.BlockSpec((1, TN, INTER), lambda n, t, te, act: (te[t], n, 0)),
            ],
            out_specs=pl.BlockSpec((TM, TN), lambda n, t, te, act: (t, n)),
        ),
        out_shape=jax.ShapeDtypeStruct((NPAD, HIDDEN), jnp.float32),
        compiler_params=pltpu.CompilerParams(
            dimension_semantics=("arbitrary", "arbitrary")),
    )(tile_expert, tile_active, h_pad, w2_weight)

    y = y_pad[pos].reshape(TOKENS, TOPK, HIDDEN)
    out = (y * topk_w[:, :, None]).sum(axis=1)
    return out.reshape(orig_shape)
